# baseline (speedup 1.0000x reference)
import functools
import jax
import jax.numpy as jnp
import numpy as np
from jax import lax
from jax.experimental import pallas as pl
from jax.experimental.pallas import tpu as pltpu
from jax.experimental.pallas import tpu_sc as plsc

_B, _W, _D, _E = 256, 25, 25, 128
_WD = _W * _D
_WDP = 640
_NC, _NS = 2, 16
_NW = _NC * _NS
_BPW = _B // _NW
_R0 = 4 * 8 * 4
_R1 = 4 * 8 * 8
_NR = _R0 + _R1
_EG = 8
_NEG = _E // _EG
_NJ = _WDP // 16


def _sc_body(tab_hbm, i0, i1, i2, i3, i4, i5, out_hbm,
             tab_v, idx_v, c_v, buf_v):
    wid = lax.axis_index("s") * _NC + lax.axis_index("c")
    b0 = wid * _BPW

    pltpu.sync_copy(tab_hbm, tab_v)
    idx_hbms = (i0, i1, i2, i3, i4, i5)
    for f in range(6):
        pltpu.sync_copy(idx_hbms[f].at[pl.ds(b0, _BPW), :], idx_v.at[f])

    def c_body(b, _):
        def cj(j, _):
            sl = pl.ds(j * 16, 16)
            c_v[0, b, sl] = (idx_v[0, b, sl] * 32 + idx_v[1, b, sl] * 4
                             + idx_v[2, b, sl])
            c_v[1, b, sl] = (idx_v[3, b, sl] * 64 + idx_v[4, b, sl] * 8
                             + idx_v[5, b, sl] + _R0)
            return 0
        return lax.fori_loop(0, _NJ, cj, 0)
    lax.fori_loop(0, _BPW, c_body, 0)

    lane0 = lax.iota(jnp.int32, 16) < 1

    def b_body(b, _):
        def eg_body(eg, _):
            def j_body(j, _):
                sl = pl.ds(j * 16, 16)
                c0 = c_v[0, b, sl]
                c1 = c_v[1, b, sl]
                for el in range(_EG):
                    base = (eg * _EG + el) * _NR
                    g0 = plsc.load_gather(tab_v, [c0 + base])
                    g1 = plsc.load_gather(tab_v, [c1 + base])
                    buf_v[pl.ds(el * _WD + j * 16, 16)] = g0 + g1
                return 0
            lax.fori_loop(0, _NJ - 1, j_body, 0)
            tsl = pl.ds((_NJ - 1) * 16, 16)
            c0 = c_v[0, b, tsl]
            c1 = c_v[1, b, tsl]
            for el in range(_EG):
                base = (eg * _EG + el) * _NR
                g0 = plsc.load_gather(tab_v, [c0 + base])
                g1 = plsc.load_gather(tab_v, [c1 + base])
                plsc.store_compressed(
                    buf_v.at[pl.ds(el * _WD + (_NJ - 1) * 16, 16)],
                    g0 + g1, mask=lane0)
            pltpu.sync_copy(buf_v.at[pl.ds(0, _EG * _WD)],
                            out_hbm.at[b0 + b, eg, :])
            return 0
        return lax.fori_loop(0, _NEG, eg_body, 0)
    lax.fori_loop(0, _BPW, b_body, 0)


def kernel(card_counts, card_colors, card_shapes, card_selections,
           leader_rotations, follower_rotations,
           T_count, T_color, T_shape, T_sel, T_lead, T_foll):
    def masked(t):
        return t.at[0].set(0.0)
    mc, mcol, msh = masked(T_count), masked(T_color), masked(T_shape)
    msel, mld, mfl = masked(T_sel), masked(T_lead), masked(T_foll)
    p0 = (mc[:, None, None, :] + mcol[None, :, None, :]
          + msh[None, None, :, :]).reshape(_R0, _E)
    p1 = (msel[:, None, None, :] + mld[None, :, None, :]
          + mfl[None, None, :, :]).reshape(_R1, _E)
    tab_flat = jnp.concatenate([p0, p1], axis=0).T.reshape(_E * _NR)

    def prep(a):
        return jnp.pad(a.reshape(_B, _WD), ((0, 0), (0, _WDP - _WD)))
    idxs = [prep(a) for a in (card_counts, card_colors, card_shapes,
                              card_selections, leader_rotations,
                              follower_rotations)]

    mesh = plsc.VectorSubcoreMesh(core_axis_name="c", subcore_axis_name="s")
    sc_fn = functools.partial(
        pl.kernel,
        mesh=mesh,
        compiler_params=pltpu.CompilerParams(use_tc_tiling_on_sc=False,
                                             needs_layout_passes=False),
        out_type=jax.ShapeDtypeStruct((_B, _NEG, _EG * _WD), jnp.float32),
        scratch_types=[
            pltpu.VMEM((_E * _NR,), jnp.float32),
            pltpu.VMEM((6, _BPW, _WDP), jnp.int32),
            pltpu.VMEM((2, _BPW, _WDP), jnp.int32),
            pltpu.VMEM((_EG * _WD + 24,), jnp.float32),
        ],
    )(_sc_body)
    out = sc_fn(tab_flat, *idxs)
    return out.reshape(_B, _E, _W, _D)


# speedup vs baseline: 5.1746x; 5.1746x over previous
import jax
import jax.numpy as jnp
import numpy as np
from jax.experimental import pallas as pl
from jax.experimental.pallas import tpu as pltpu

_B = 256
_W = 25
_D = 25
_WD = _W * _D
_E = 128
_VOCAB_SIZES = (4, 8, 4, 4, 8, 8)
_NROWS = sum(_VOCAB_SIZES)
_OFFSETS = tuple(int(x) for x in np.cumsum((0,) + _VOCAB_SIZES[:-1]))

_B_BLK = 64


def _embed_body(i0, i1, i2, i3, i4, i5, tabT_ref, out_ref):
    tabT = tabT_ref[...]
    idx_refs = (i0, i1, i2, i3, i4, i5)
    rows = jax.lax.broadcasted_iota(jnp.int32, (_NROWS, _WD), 0)
    for bb in range(_B_BLK):
        oh = jnp.zeros((_NROWS, _WD), dtype=jnp.float32)
        for f in range(6):
            idx_f = idx_refs[f][bb, 0] + _OFFSETS[f]
            oh = oh + (rows == idx_f[None, :]).astype(jnp.float32)
        out_ref[bb] = jnp.dot(tabT, oh, preferred_element_type=jnp.float32)


def kernel(card_counts, card_colors, card_shapes, card_selections,
           leader_rotations, follower_rotations,
           T_count, T_color, T_shape, T_sel, T_lead, T_foll):
    idx_arrays = [a.reshape(_B, 1, _WD) for a in
                  (card_counts, card_colors, card_shapes, card_selections,
                   leader_rotations, follower_rotations)]

    tab = jnp.concatenate([T_count, T_color, T_shape, T_sel, T_lead, T_foll],
                          axis=0)
    row_mask = np.ones((_NROWS, 1), dtype=np.float32)
    for off in _OFFSETS:
        row_mask[off, 0] = 0.0
    tabT = (tab * jnp.asarray(row_mask)).T

    idx_spec = pl.BlockSpec((_B_BLK, 1, _WD), lambda i: (i, 0, 0))
    out = pl.pallas_call(
        _embed_body,
        grid=(_B // _B_BLK,),
        in_specs=[idx_spec] * 6 + [pl.BlockSpec((_E, _NROWS), lambda i: (0, 0))],
        out_specs=pl.BlockSpec((_B_BLK, _E, _WD), lambda i: (i, 0, 0)),
        out_shape=jax.ShapeDtypeStruct((_B, _E, _WD), jnp.float32),
        compiler_params=pltpu.CompilerParams(
            dimension_semantics=("parallel",)),
    )(*idx_arrays, tabT)
    return out.reshape(_B, _E, _W, _D)


# speedup vs baseline: 5.2469x; 1.0140x over previous
import jax
import jax.numpy as jnp
import numpy as np
from jax.experimental import pallas as pl
from jax.experimental.pallas import tpu as pltpu

_B = 256
_W = 25
_D = 25
_WD = _W * _D
_E = 128
_VOCAB_SIZES = (4, 8, 4, 4, 8, 8)
_NROWS = sum(_VOCAB_SIZES)
_OFFSETS = tuple(int(x) for x in np.cumsum((0,) + _VOCAB_SIZES[:-1]))

_B_BLK = 32


def _embed_body(i0, i1, i2, i3, i4, i5, tabT_ref, out_ref):
    tabT = tabT_ref[...]
    idx_refs = (i0, i1, i2, i3, i4, i5)
    rows = jax.lax.broadcasted_iota(jnp.int32, (_NROWS, _WD), 0)
    for bb in range(_B_BLK):
        oh = jnp.zeros((_NROWS, _WD), dtype=jnp.float32)
        for f in range(6):
            idx_f = idx_refs[f][bb, 0] + _OFFSETS[f]
            oh = oh + (rows == idx_f[None, :]).astype(jnp.float32)
        out_ref[bb] = jnp.dot(tabT, oh, preferred_element_type=jnp.float32)


def kernel(card_counts, card_colors, card_shapes, card_selections,
           leader_rotations, follower_rotations,
           T_count, T_color, T_shape, T_sel, T_lead, T_foll):
    idx_arrays = [a.reshape(_B, 1, _WD) for a in
                  (card_counts, card_colors, card_shapes, card_selections,
                   leader_rotations, follower_rotations)]

    tab = jnp.concatenate([T_count, T_color, T_shape, T_sel, T_lead, T_foll],
                          axis=0)
    row_mask = np.ones((_NROWS, 1), dtype=np.float32)
    for off in _OFFSETS:
        row_mask[off, 0] = 0.0
    tabT = (tab * jnp.asarray(row_mask)).T

    idx_spec = pl.BlockSpec((_B_BLK, 1, _WD), lambda i: (i, 0, 0))
    out = pl.pallas_call(
        _embed_body,
        grid=(_B // _B_BLK,),
        in_specs=[idx_spec] * 6 + [pl.BlockSpec((_E, _NROWS), lambda i: (0, 0))],
        out_specs=pl.BlockSpec((_B_BLK, _E, _WD), lambda i: (i, 0, 0)),
        out_shape=jax.ShapeDtypeStruct((_B, _E, _WD), jnp.float32),
        compiler_params=pltpu.CompilerParams(
            dimension_semantics=("parallel",)),
    )(*idx_arrays, tabT)
    return out.reshape(_B, _E, _W, _D)
